# Initial kernel scaffold; baseline (speedup 1.0000x reference)
import functools

import jax
import jax.numpy as jnp
from jax import lax
from jax.experimental import pallas as pl
from jax.experimental.pallas import tpu as pltpu
from jax.experimental.pallas import tpu_sc as plsc

N_NODES = 10000
D = 256
HALF = 128
N_PAD = 10240
E_PAD = 163840
NC = 2
NS = 16
NTILES = NC * NS
BLK = 128
NBLK = E_PAD // (NTILES * BLK)
RPT = N_PAD // NS
RB = 640


def _sc_degree(dst_blocks, zeros16, ones16):
    mesh = plsc.VectorSubcoreMesh(core_axis_name="c", subcore_axis_name="s")

    @functools.partial(
        pl.kernel,
        mesh=mesh,
        out_type=jax.ShapeDtypeStruct((NC, N_PAD, 16), jnp.float32),
        scratch_types=[
            pltpu.VMEM((NBLK, BLK), jnp.int32),
            pltpu.VMEM((BLK, 16), jnp.float32),
            pltpu.VMEM_SHARED((N_PAD, 16), jnp.float32),
        ],
    )
    def k(dst_r, z_r, o_r, out_r, didx, ones_v, acc):
        cid = lax.axis_index("c")
        sid = lax.axis_index("s")
        wid = cid * NS + sid
        rows = pl.ds(sid * RPT, RPT)
        pltpu.sync_copy(z_r.at[rows], acc.at[rows])
        pltpu.sync_copy(o_r, ones_v)
        pltpu.sync_copy(dst_r.at[wid], didx)
        plsc.subcore_barrier()

        def body(j, carry):
            pltpu.sync_copy(ones_v, acc.at[didx.at[j]], add=True)
            return carry

        lax.fori_loop(0, NBLK, body, 0)
        plsc.subcore_barrier()
        pltpu.sync_copy(acc.at[rows], out_r.at[cid].at[rows])

    return k(dst_blocks, zeros16, ones16)


def _sc_propagate(t0, t1, src_blocks, dst_blocks, zeros128):
    mesh = plsc.VectorSubcoreMesh(core_axis_name="c", subcore_axis_name="s")

    @functools.partial(
        pl.kernel,
        mesh=mesh,
        out_type=jax.ShapeDtypeStruct((NC, 2, N_PAD, HALF), jnp.float32),
        scratch_types=[
            pltpu.VMEM((NBLK, BLK), jnp.int32),
            pltpu.VMEM((NBLK, BLK), jnp.int32),
            pltpu.VMEM((BLK, HALF), jnp.float32),
            pltpu.VMEM_SHARED((N_PAD, HALF), jnp.float32),
            pltpu.SemaphoreType.DMA,
        ],
    )
    def k(t0_r, t1_r, src_r, dst_r, z_r, out_r, sidx, didx, rows_v, acc, sem):
        cid = lax.axis_index("c")
        sid = lax.axis_index("s")
        wid = cid * NS + sid
        rows = pl.ds(sid * RPT, RPT)
        pltpu.sync_copy(src_r.at[wid], sidx)
        pltpu.sync_copy(dst_r.at[wid], didx)
        for half, t_r in ((0, t0_r), (1, t1_r)):
            pltpu.sync_copy(z_r.at[rows], acc.at[rows])
            plsc.subcore_barrier()

            def body(j, carry):
                pltpu.async_copy(t_r.at[sidx.at[j]], rows_v, sem).wait()
                pltpu.sync_copy(rows_v, acc.at[didx.at[j]], add=True)
                return carry

            lax.fori_loop(0, NBLK, body, 0)
            plsc.subcore_barrier()
            pltpu.sync_copy(acc.at[rows], out_r.at[cid, half].at[rows])

    return k(t0, t1, src_blocks, dst_blocks, zeros128)


def _dinv(d0_r, d1_r):
    deg = d0_r[:, :1] + d1_r[:, :1] + 1.0
    return lax.rsqrt(deg)


def _tc_stage1(x, W1, d0, d1):
    def body(x_r, w_r, d0_r, d1_r, h_r, t0_r, t1_r):
        dinv = _dinv(d0_r, d1_r)
        h = jnp.dot(x_r[...], w_r[...], preferred_element_type=jnp.float32)
        h_r[...] = h
        hs = h * dinv
        t0_r[...] = hs[:, :HALF]
        t1_r[...] = hs[:, HALF:]

    return pl.pallas_call(
        body,
        grid=(N_PAD // RB,),
        in_specs=[
            pl.BlockSpec((RB, D), lambda i: (i, 0)),
            pl.BlockSpec((D, D), lambda i: (0, 0)),
            pl.BlockSpec((RB, 16), lambda i: (i, 0)),
            pl.BlockSpec((RB, 16), lambda i: (i, 0)),
        ],
        out_specs=[
            pl.BlockSpec((RB, D), lambda i: (i, 0)),
            pl.BlockSpec((RB, HALF), lambda i: (i, 0)),
            pl.BlockSpec((RB, HALF), lambda i: (i, 0)),
        ],
        out_shape=[
            jax.ShapeDtypeStruct((N_PAD, D), jnp.float32),
            jax.ShapeDtypeStruct((N_PAD, HALF), jnp.float32),
            jax.ShapeDtypeStruct((N_PAD, HALF), jnp.float32),
        ],
    )(x, W1, d0, d1)


def _tc_stage2(a0, a1, h1, W2, b1, d0, d1):
    def body(a0_r, a1_r, h1_r, w_r, b_r, d0_r, d1_r, h2_r, u0_r, u1_r):
        dinv = _dinv(d0_r, d1_r)
        z = (a0_r[...] + a1_r[...]) * dinv + (dinv * dinv) * h1_r[...] + b_r[...]
        z = jnp.maximum(z, 0.0)
        h2 = jnp.dot(z, w_r[...], preferred_element_type=jnp.float32)
        h2_r[...] = h2
        hs = h2 * dinv
        u0_r[...] = hs[:, :HALF]
        u1_r[...] = hs[:, HALF:]

    return pl.pallas_call(
        body,
        grid=(N_PAD // RB,),
        in_specs=[
            pl.BlockSpec((RB, D), lambda i: (i, 0)),
            pl.BlockSpec((RB, D), lambda i: (i, 0)),
            pl.BlockSpec((RB, D), lambda i: (i, 0)),
            pl.BlockSpec((D, D), lambda i: (0, 0)),
            pl.BlockSpec((1, D), lambda i: (0, 0)),
            pl.BlockSpec((RB, 16), lambda i: (i, 0)),
            pl.BlockSpec((RB, 16), lambda i: (i, 0)),
        ],
        out_specs=[
            pl.BlockSpec((RB, D), lambda i: (i, 0)),
            pl.BlockSpec((RB, HALF), lambda i: (i, 0)),
            pl.BlockSpec((RB, HALF), lambda i: (i, 0)),
        ],
        out_shape=[
            jax.ShapeDtypeStruct((N_PAD, D), jnp.float32),
            jax.ShapeDtypeStruct((N_PAD, HALF), jnp.float32),
            jax.ShapeDtypeStruct((N_PAD, HALF), jnp.float32),
        ],
    )(a0, a1, h1, W2, b1, d0, d1)


def _tc_stage3(c0, c1, h2, b2, d0, d1):
    def body(c0_r, c1_r, h2_r, b_r, d0_r, d1_r, o_r):
        dinv = _dinv(d0_r, d1_r)
        o_r[...] = (
            (c0_r[...] + c1_r[...]) * dinv
            + (dinv * dinv) * h2_r[...]
            + b_r[...]
        )

    return pl.pallas_call(
        body,
        grid=(N_PAD // RB,),
        in_specs=[
            pl.BlockSpec((RB, D), lambda i: (i, 0)),
            pl.BlockSpec((RB, D), lambda i: (i, 0)),
            pl.BlockSpec((RB, D), lambda i: (i, 0)),
            pl.BlockSpec((1, D), lambda i: (0, 0)),
            pl.BlockSpec((RB, 16), lambda i: (i, 0)),
            pl.BlockSpec((RB, 16), lambda i: (i, 0)),
        ],
        out_specs=pl.BlockSpec((RB, D), lambda i: (i, 0)),
        out_shape=jax.ShapeDtypeStruct((N_PAD, D), jnp.float32),
    )(c0, c1, h2, b2, d0, d1)


def kernel(x, edge_index, W1, b1, W2, b2):
    src = edge_index[0].astype(jnp.int32)
    dst = edge_index[1].astype(jnp.int32)
    e = src.shape[0]
    padfill = jnp.full((E_PAD - e,), N_NODES, jnp.int32)
    srcb = jnp.concatenate([src, padfill]).reshape(NTILES, NBLK, BLK)
    dstb = jnp.concatenate([dst, padfill]).reshape(NTILES, NBLK, BLK)
    xp = jnp.zeros((N_PAD, D), jnp.float32).at[:N_NODES].set(x)
    z16 = jnp.zeros((N_PAD, 16), jnp.float32)
    o16 = jnp.ones((BLK, 16), jnp.float32)
    z128 = jnp.zeros((N_PAD, HALF), jnp.float32)

    degp = _sc_degree(dstb, z16, o16)
    d0, d1 = degp[0], degp[1]

    h1, t0, t1 = _tc_stage1(xp, W1, d0, d1)
    p = _sc_propagate(t0, t1, srcb, dstb, z128)
    a0 = jnp.concatenate([p[0, 0], p[0, 1]], axis=1)
    a1 = jnp.concatenate([p[1, 0], p[1, 1]], axis=1)

    h2, u0, u1 = _tc_stage2(a0, a1, h1, W2, b1.reshape(1, D), d0, d1)
    q = _sc_propagate(u0, u1, srcb, dstb, z128)
    c0 = jnp.concatenate([q[0, 0], q[0, 1]], axis=1)
    c1 = jnp.concatenate([q[1, 0], q[1, 1]], axis=1)

    out = _tc_stage3(c0, c1, h2, b2.reshape(1, D), d0, d1)
    return out[:N_NODES]


# speedup vs baseline: 4.5461x; 4.5461x over previous
import functools

import jax
import jax.numpy as jnp
from jax import lax
from jax.experimental import pallas as pl
from jax.experimental.pallas import tpu as pltpu
from jax.experimental.pallas import tpu_sc as plsc

N_NODES = 10000
D = 256
HALF = 128
N_PAD = 10240
E_PAD = 163840
NC = 2
NS = 16
NTILES = NC * NS
BLK = 128
NBLK = E_PAD // (NTILES * BLK)
RPT = N_PAD // NS
RB = 640


def _sc_degree(dst_blocks, zeros128, ones128):
    mesh = plsc.VectorSubcoreMesh(core_axis_name="c", subcore_axis_name="s")

    @functools.partial(
        pl.kernel,
        mesh=mesh,
        out_type=jax.ShapeDtypeStruct((NC, N_PAD, HALF), jnp.float32),
        scratch_types=[
            pltpu.VMEM((NBLK, BLK), jnp.int32),
            pltpu.VMEM((BLK, HALF), jnp.float32),
            pltpu.VMEM_SHARED((N_PAD, HALF), jnp.float32),
        ],
    )
    def k(dst_r, z_r, o_r, out_r, didx, ones_v, acc):
        cid = lax.axis_index("c")
        sid = lax.axis_index("s")
        wid = cid * NS + sid
        rows = pl.ds(sid * RPT, RPT)
        pltpu.sync_copy(z_r.at[rows], acc.at[rows])
        pltpu.sync_copy(o_r, ones_v)
        pltpu.sync_copy(dst_r.at[wid], didx)
        plsc.subcore_barrier()

        def body(j, carry):
            pltpu.sync_copy(ones_v, acc.at[didx.at[j]], add=True)
            return carry

        lax.fori_loop(0, NBLK, body, 0)
        plsc.subcore_barrier()
        pltpu.sync_copy(acc.at[rows], out_r.at[cid].at[rows])

    return k(dst_blocks, zeros128, ones128)


def _sc_propagate(t0, t1, src_blocks, dst_blocks, zeros128):
    mesh = plsc.VectorSubcoreMesh(core_axis_name="c", subcore_axis_name="s")

    @functools.partial(
        pl.kernel,
        mesh=mesh,
        out_type=jax.ShapeDtypeStruct((NC, 2, N_PAD, HALF), jnp.float32),
        scratch_types=[
            pltpu.VMEM((NBLK, BLK), jnp.int32),
            pltpu.VMEM((NBLK, BLK), jnp.int32),
            pltpu.VMEM((BLK, HALF), jnp.float32),
            pltpu.VMEM_SHARED((N_PAD, HALF), jnp.float32),
            pltpu.SemaphoreType.DMA,
        ],
    )
    def k(t0_r, t1_r, src_r, dst_r, z_r, out_r, sidx, didx, rows_v, acc, sem):
        cid = lax.axis_index("c")
        sid = lax.axis_index("s")
        wid = cid * NS + sid
        rows = pl.ds(sid * RPT, RPT)
        pltpu.sync_copy(src_r.at[wid], sidx)
        pltpu.sync_copy(dst_r.at[wid], didx)
        for half, t_r in ((0, t0_r), (1, t1_r)):
            pltpu.sync_copy(z_r.at[rows], acc.at[rows])
            plsc.subcore_barrier()

            def body(j, carry):
                pltpu.async_copy(t_r.at[sidx.at[j]], rows_v, sem).wait()
                pltpu.sync_copy(rows_v, acc.at[didx.at[j]], add=True)
                return carry

            lax.fori_loop(0, NBLK, body, 0)
            plsc.subcore_barrier()
            pltpu.sync_copy(acc.at[rows], out_r.at[cid, half].at[rows])

    return k(t0, t1, src_blocks, dst_blocks, zeros128)


def _dinv(d0_r, d1_r):
    deg = d0_r[:, :1] + d1_r[:, :1] + 1.0
    return lax.rsqrt(deg)


def _tc_stage1(x, W1, d0, d1):
    def body(x_r, w_r, d0_r, d1_r, h_r, t0_r, t1_r):
        dinv = _dinv(d0_r, d1_r)
        h = jnp.dot(x_r[...], w_r[...], preferred_element_type=jnp.float32)
        h_r[...] = h
        hs = h * dinv
        t0_r[...] = hs[:, :HALF]
        t1_r[...] = hs[:, HALF:]

    return pl.pallas_call(
        body,
        grid=(N_PAD // RB,),
        in_specs=[
            pl.BlockSpec((RB, D), lambda i: (i, 0)),
            pl.BlockSpec((D, D), lambda i: (0, 0)),
            pl.BlockSpec((RB, 16), lambda i: (i, 0)),
            pl.BlockSpec((RB, 16), lambda i: (i, 0)),
        ],
        out_specs=[
            pl.BlockSpec((RB, D), lambda i: (i, 0)),
            pl.BlockSpec((RB, HALF), lambda i: (i, 0)),
            pl.BlockSpec((RB, HALF), lambda i: (i, 0)),
        ],
        out_shape=[
            jax.ShapeDtypeStruct((N_PAD, D), jnp.float32),
            jax.ShapeDtypeStruct((N_PAD, HALF), jnp.float32),
            jax.ShapeDtypeStruct((N_PAD, HALF), jnp.float32),
        ],
    )(x, W1, d0, d1)


def _tc_stage2(a0, a1, h1, W2, b1, d0, d1):
    def body(a0_r, a1_r, h1_r, w_r, b_r, d0_r, d1_r, h2_r, u0_r, u1_r):
        dinv = _dinv(d0_r, d1_r)
        z = (a0_r[...] + a1_r[...]) * dinv + (dinv * dinv) * h1_r[...] + b_r[...]
        z = jnp.maximum(z, 0.0)
        h2 = jnp.dot(z, w_r[...], preferred_element_type=jnp.float32)
        h2_r[...] = h2
        hs = h2 * dinv
        u0_r[...] = hs[:, :HALF]
        u1_r[...] = hs[:, HALF:]

    return pl.pallas_call(
        body,
        grid=(N_PAD // RB,),
        in_specs=[
            pl.BlockSpec((RB, D), lambda i: (i, 0)),
            pl.BlockSpec((RB, D), lambda i: (i, 0)),
            pl.BlockSpec((RB, D), lambda i: (i, 0)),
            pl.BlockSpec((D, D), lambda i: (0, 0)),
            pl.BlockSpec((1, D), lambda i: (0, 0)),
            pl.BlockSpec((RB, 16), lambda i: (i, 0)),
            pl.BlockSpec((RB, 16), lambda i: (i, 0)),
        ],
        out_specs=[
            pl.BlockSpec((RB, D), lambda i: (i, 0)),
            pl.BlockSpec((RB, HALF), lambda i: (i, 0)),
            pl.BlockSpec((RB, HALF), lambda i: (i, 0)),
        ],
        out_shape=[
            jax.ShapeDtypeStruct((N_PAD, D), jnp.float32),
            jax.ShapeDtypeStruct((N_PAD, HALF), jnp.float32),
            jax.ShapeDtypeStruct((N_PAD, HALF), jnp.float32),
        ],
    )(a0, a1, h1, W2, b1, d0, d1)


def _tc_stage3(c0, c1, h2, b2, d0, d1):
    def body(c0_r, c1_r, h2_r, b_r, d0_r, d1_r, o_r):
        dinv = _dinv(d0_r, d1_r)
        o_r[...] = (
            (c0_r[...] + c1_r[...]) * dinv
            + (dinv * dinv) * h2_r[...]
            + b_r[...]
        )

    return pl.pallas_call(
        body,
        grid=(N_PAD // RB,),
        in_specs=[
            pl.BlockSpec((RB, D), lambda i: (i, 0)),
            pl.BlockSpec((RB, D), lambda i: (i, 0)),
            pl.BlockSpec((RB, D), lambda i: (i, 0)),
            pl.BlockSpec((1, D), lambda i: (0, 0)),
            pl.BlockSpec((RB, 16), lambda i: (i, 0)),
            pl.BlockSpec((RB, 16), lambda i: (i, 0)),
        ],
        out_specs=pl.BlockSpec((RB, D), lambda i: (i, 0)),
        out_shape=jax.ShapeDtypeStruct((N_PAD, D), jnp.float32),
    )(c0, c1, h2, b2, d0, d1)


def kernel(x, edge_index, W1, b1, W2, b2):
    src = edge_index[0].astype(jnp.int32)
    dst = edge_index[1].astype(jnp.int32)
    e = src.shape[0]
    padfill = jnp.full((E_PAD - e,), N_NODES, jnp.int32)
    srcb = jnp.concatenate([src, padfill]).reshape(NTILES, NBLK, BLK)
    dstb = jnp.concatenate([dst, padfill]).reshape(NTILES, NBLK, BLK)
    xp = jnp.zeros((N_PAD, D), jnp.float32).at[:N_NODES].set(x)
    z128 = jnp.zeros((N_PAD, HALF), jnp.float32)
    o128 = jnp.ones((BLK, HALF), jnp.float32)

    degp = _sc_degree(dstb, z128, o128)
    d0, d1 = degp[0, :, :16], degp[1, :, :16]

    h1, t0, t1 = _tc_stage1(xp, W1, d0, d1)
    p = _sc_propagate(t0, t1, srcb, dstb, z128)
    a0 = jnp.concatenate([p[0, 0], p[0, 1]], axis=1)
    a1 = jnp.concatenate([p[1, 0], p[1, 1]], axis=1)

    h2, u0, u1 = _tc_stage2(a0, a1, h1, W2, b1.reshape(1, D), d0, d1)
    q = _sc_propagate(u0, u1, srcb, dstb, z128)
    c0 = jnp.concatenate([q[0, 0], q[0, 1]], axis=1)
    c1 = jnp.concatenate([q[1, 0], q[1, 1]], axis=1)

    out = _tc_stage3(c0, c1, h2, b2.reshape(1, D), d0, d1)
    return out[:N_NODES]


# speedup vs baseline: 5.0694x; 1.1151x over previous
import functools

import jax
import jax.numpy as jnp
from jax import lax
from jax.experimental import pallas as pl
from jax.experimental.pallas import tpu as pltpu
from jax.experimental.pallas import tpu_sc as plsc

N_NODES = 10000
D = 256
HALF = 128
N_PAD = 10240
E_PAD = 163840
NC = 2
NS = 16
NTILES = NC * NS
BLK = 128
NBLK = E_PAD // (NTILES * BLK)
RPT = N_PAD // NS
RB = 640


def _sc_degree(dst_blocks, zeros128, ones128):
    mesh = plsc.VectorSubcoreMesh(core_axis_name="c", subcore_axis_name="s")

    @functools.partial(
        pl.kernel,
        mesh=mesh,
        out_type=jax.ShapeDtypeStruct((NC, N_PAD, HALF), jnp.float32),
        scratch_types=[
            pltpu.VMEM((NBLK, BLK), jnp.int32),
            pltpu.VMEM((BLK, HALF), jnp.float32),
            pltpu.VMEM_SHARED((N_PAD, HALF), jnp.float32),
        ],
    )
    def k(dst_r, z_r, o_r, out_r, didx, ones_v, acc):
        cid = lax.axis_index("c")
        sid = lax.axis_index("s")
        wid = cid * NS + sid
        rows = pl.ds(sid * RPT, RPT)
        pltpu.sync_copy(z_r.at[rows], acc.at[rows])
        pltpu.sync_copy(o_r, ones_v)
        pltpu.sync_copy(dst_r.at[wid], didx)
        plsc.subcore_barrier()

        def body(j, carry):
            pltpu.sync_copy(ones_v, acc.at[didx.at[j]], add=True)
            return carry

        lax.fori_loop(0, NBLK, body, 0)
        plsc.subcore_barrier()
        pltpu.sync_copy(acc.at[rows], out_r.at[cid].at[rows])

    return k(dst_blocks, zeros128, ones128)


def _sc_propagate(t0, t1, src_blocks, dst_blocks, zeros128):
    mesh = plsc.VectorSubcoreMesh(core_axis_name="c", subcore_axis_name="s")

    @functools.partial(
        pl.kernel,
        mesh=mesh,
        out_type=jax.ShapeDtypeStruct((NC, 2, N_PAD, HALF), jnp.float32),
        scratch_types=[
            pltpu.VMEM((NBLK, BLK), jnp.int32),
            pltpu.VMEM((NBLK, BLK), jnp.int32),
            pltpu.VMEM((2, BLK, HALF), jnp.float32),
            pltpu.VMEM_SHARED((N_PAD, HALF), jnp.float32),
            pltpu.SemaphoreType.DMA,
        ],
    )
    def k(t0_r, t1_r, src_r, dst_r, z_r, out_r, sidx, didx, rows_v, acc, sem):
        cid = lax.axis_index("c")
        sid = lax.axis_index("s")
        wid = cid * NS + sid
        rows = pl.ds(sid * RPT, RPT)
        pltpu.sync_copy(src_r.at[wid], sidx)
        pltpu.sync_copy(dst_r.at[wid], didx)
        for half, t_r in ((0, t0_r), (1, t1_r)):
            pltpu.sync_copy(z_r.at[rows], acc.at[rows])
            plsc.subcore_barrier()
            pltpu.async_copy(t_r.at[sidx.at[0]], rows_v.at[0], sem)

            def body(j, carry):
                b = j % 2

                @pl.when(j + 1 < NBLK)
                def _():
                    pltpu.async_copy(
                        t_r.at[sidx.at[j + 1]], rows_v.at[1 - b], sem
                    )

                pltpu.make_async_copy(
                    t_r.at[sidx.at[j]], rows_v.at[b], sem
                ).wait()
                pltpu.sync_copy(rows_v.at[b], acc.at[didx.at[j]], add=True)
                return carry

            lax.fori_loop(0, NBLK, body, 0)
            plsc.subcore_barrier()
            pltpu.sync_copy(acc.at[rows], out_r.at[cid, half].at[rows])

    return k(t0, t1, src_blocks, dst_blocks, zeros128)


def _dinv(d0_r, d1_r):
    deg = d0_r[:, :1] + d1_r[:, :1] + 1.0
    return lax.rsqrt(deg)


def _tc_stage1(x, W1, d0, d1):
    def body(x_r, w_r, d0_r, d1_r, h_r, t0_r, t1_r):
        dinv = _dinv(d0_r, d1_r)
        h = jnp.dot(x_r[...], w_r[...], preferred_element_type=jnp.float32)
        h_r[...] = h
        hs = h * dinv
        t0_r[...] = hs[:, :HALF]
        t1_r[...] = hs[:, HALF:]

    return pl.pallas_call(
        body,
        grid=(N_PAD // RB,),
        in_specs=[
            pl.BlockSpec((RB, D), lambda i: (i, 0)),
            pl.BlockSpec((D, D), lambda i: (0, 0)),
            pl.BlockSpec((RB, 16), lambda i: (i, 0)),
            pl.BlockSpec((RB, 16), lambda i: (i, 0)),
        ],
        out_specs=[
            pl.BlockSpec((RB, D), lambda i: (i, 0)),
            pl.BlockSpec((RB, HALF), lambda i: (i, 0)),
            pl.BlockSpec((RB, HALF), lambda i: (i, 0)),
        ],
        out_shape=[
            jax.ShapeDtypeStruct((N_PAD, D), jnp.float32),
            jax.ShapeDtypeStruct((N_PAD, HALF), jnp.float32),
            jax.ShapeDtypeStruct((N_PAD, HALF), jnp.float32),
        ],
    )(x, W1, d0, d1)


def _tc_stage2(a0, a1, h1, W2, b1, d0, d1):
    def body(a0_r, a1_r, h1_r, w_r, b_r, d0_r, d1_r, h2_r, u0_r, u1_r):
        dinv = _dinv(d0_r, d1_r)
        z = (a0_r[...] + a1_r[...]) * dinv + (dinv * dinv) * h1_r[...] + b_r[...]
        z = jnp.maximum(z, 0.0)
        h2 = jnp.dot(z, w_r[...], preferred_element_type=jnp.float32)
        h2_r[...] = h2
        hs = h2 * dinv
        u0_r[...] = hs[:, :HALF]
        u1_r[...] = hs[:, HALF:]

    return pl.pallas_call(
        body,
        grid=(N_PAD // RB,),
        in_specs=[
            pl.BlockSpec((RB, D), lambda i: (i, 0)),
            pl.BlockSpec((RB, D), lambda i: (i, 0)),
            pl.BlockSpec((RB, D), lambda i: (i, 0)),
            pl.BlockSpec((D, D), lambda i: (0, 0)),
            pl.BlockSpec((1, D), lambda i: (0, 0)),
            pl.BlockSpec((RB, 16), lambda i: (i, 0)),
            pl.BlockSpec((RB, 16), lambda i: (i, 0)),
        ],
        out_specs=[
            pl.BlockSpec((RB, D), lambda i: (i, 0)),
            pl.BlockSpec((RB, HALF), lambda i: (i, 0)),
            pl.BlockSpec((RB, HALF), lambda i: (i, 0)),
        ],
        out_shape=[
            jax.ShapeDtypeStruct((N_PAD, D), jnp.float32),
            jax.ShapeDtypeStruct((N_PAD, HALF), jnp.float32),
            jax.ShapeDtypeStruct((N_PAD, HALF), jnp.float32),
        ],
    )(a0, a1, h1, W2, b1, d0, d1)


def _tc_stage3(c0, c1, h2, b2, d0, d1):
    def body(c0_r, c1_r, h2_r, b_r, d0_r, d1_r, o_r):
        dinv = _dinv(d0_r, d1_r)
        o_r[...] = (
            (c0_r[...] + c1_r[...]) * dinv
            + (dinv * dinv) * h2_r[...]
            + b_r[...]
        )

    return pl.pallas_call(
        body,
        grid=(N_PAD // RB,),
        in_specs=[
            pl.BlockSpec((RB, D), lambda i: (i, 0)),
            pl.BlockSpec((RB, D), lambda i: (i, 0)),
            pl.BlockSpec((RB, D), lambda i: (i, 0)),
            pl.BlockSpec((1, D), lambda i: (0, 0)),
            pl.BlockSpec((RB, 16), lambda i: (i, 0)),
            pl.BlockSpec((RB, 16), lambda i: (i, 0)),
        ],
        out_specs=pl.BlockSpec((RB, D), lambda i: (i, 0)),
        out_shape=jax.ShapeDtypeStruct((N_PAD, D), jnp.float32),
    )(c0, c1, h2, b2, d0, d1)


def kernel(x, edge_index, W1, b1, W2, b2):
    src = edge_index[0].astype(jnp.int32)
    dst = edge_index[1].astype(jnp.int32)
    e = src.shape[0]
    padfill = jnp.full((E_PAD - e,), N_NODES, jnp.int32)
    srcb = jnp.concatenate([src, padfill]).reshape(NTILES, NBLK, BLK)
    dstb = jnp.concatenate([dst, padfill]).reshape(NTILES, NBLK, BLK)
    xp = jnp.zeros((N_PAD, D), jnp.float32).at[:N_NODES].set(x)
    z128 = jnp.zeros((N_PAD, HALF), jnp.float32)
    o128 = jnp.ones((BLK, HALF), jnp.float32)

    degp = _sc_degree(dstb, z128, o128)
    d0, d1 = degp[0, :, :16], degp[1, :, :16]

    h1, t0, t1 = _tc_stage1(xp, W1, d0, d1)
    p = _sc_propagate(t0, t1, srcb, dstb, z128)
    a0 = jnp.concatenate([p[0, 0], p[0, 1]], axis=1)
    a1 = jnp.concatenate([p[1, 0], p[1, 1]], axis=1)

    h2, u0, u1 = _tc_stage2(a0, a1, h1, W2, b1.reshape(1, D), d0, d1)
    q = _sc_propagate(u0, u1, srcb, dstb, z128)
    c0 = jnp.concatenate([q[0, 0], q[0, 1]], axis=1)
    c1 = jnp.concatenate([q[1, 0], q[1, 1]], axis=1)

    out = _tc_stage3(c0, c1, h2, b2.reshape(1, D), d0, d1)
    return out[:N_NODES]
